# Initial kernel scaffold; baseline (speedup 1.0000x reference)
import jax
import jax.numpy as jnp
from jax.experimental import pallas as pl


def kernel(input, embeddings):
    raise NotImplementedError("write your pallas kernel here")


# speedup vs baseline: 1.3275x; 1.3275x over previous
import jax
import jax.numpy as jnp
from jax.experimental import pallas as pl

DIM = 64
N_EMBED = 1024
B, H, W = 16, 32, 32
NTOK = H * W
TOTAL = B * DIM * H * W


def _vq_body(x_ref, e_ref, q_ref, ind_ref, dsum_ref):
    b = pl.program_id(0)
    x = x_ref[0]
    e = e_ref[...]

    xt = x.T
    x2 = jnp.sum(xt * xt, axis=1, keepdims=True)
    e2 = jnp.sum(e * e, axis=0, keepdims=True)
    prod = jnp.dot(xt, e, preferred_element_type=jnp.float32)
    dist = x2 - 2.0 * prod + e2
    ind = jnp.argmin(dist, axis=1).astype(jnp.int32)

    code_ids = jax.lax.broadcasted_iota(jnp.int32, (N_EMBED, NTOK), 0)
    onehot = (code_ids == ind[None, :]).astype(jnp.float32)
    q = jax.lax.dot(e, onehot, precision=jax.lax.Precision.HIGHEST,
                    preferred_element_type=jnp.float32)

    q_ref[0] = q
    ind_ref[0, 0] = ind

    part = jnp.sum((q - x) ** 2).reshape(1, 1)

    @pl.when(b == 0)
    def _init():
        dsum_ref[...] = jnp.zeros((1, 1), jnp.float32)

    dsum_ref[...] += part

    @pl.when(b == B - 1)
    def _finish():
        dsum_ref[...] = dsum_ref[...] / TOTAL


def kernel(input, embeddings):
    x = input.reshape(B, DIM, NTOK)
    q, ind, dsum = pl.pallas_call(
        _vq_body,
        grid=(B,),
        in_specs=[
            pl.BlockSpec((1, DIM, NTOK), lambda b: (b, 0, 0)),
            pl.BlockSpec((DIM, N_EMBED), lambda b: (0, 0)),
        ],
        out_specs=[
            pl.BlockSpec((1, DIM, NTOK), lambda b: (b, 0, 0)),
            pl.BlockSpec((1, 1, NTOK), lambda b: (b, 0, 0)),
            pl.BlockSpec((1, 1), lambda b: (0, 0)),
        ],
        out_shape=[
            jax.ShapeDtypeStruct((B, DIM, NTOK), jnp.float32),
            jax.ShapeDtypeStruct((B, 1, NTOK), jnp.int32),
            jax.ShapeDtypeStruct((1, 1), jnp.float32),
        ],
    )(x, embeddings)
    quantize_st = q.reshape(B, DIM, H, W)
    embed_ind = ind.reshape(B, H, W)
    diff = dsum.reshape(())
    return quantize_st, diff, embed_ind


# speedup vs baseline: 2.0072x; 1.5121x over previous
import jax
import jax.numpy as jnp
from jax.experimental import pallas as pl

DIM = 64
N_EMBED = 1024
B, H, W = 16, 32, 32
NTOK = H * W
TOTAL = B * DIM * H * W


def _vq_body(x_ref, e_ref, q_ref, ind_ref, dsum_ref):
    b = pl.program_id(0)
    x = x_ref[0]
    e = e_ref[...]

    xt = x.T
    x2 = jnp.sum(xt * xt, axis=1, keepdims=True)
    e2 = jnp.sum(e * e, axis=0, keepdims=True)
    prod = jnp.dot(xt, e, preferred_element_type=jnp.float32)
    dist = x2 - 2.0 * prod + e2
    ind = jnp.argmin(dist, axis=1).astype(jnp.int32)

    code_ids = jax.lax.broadcasted_iota(jnp.int32, (N_EMBED, NTOK), 0)
    onehot = (code_ids == ind[None, :]).astype(jnp.float32)
    q = jax.lax.dot(e, onehot,
                    preferred_element_type=jnp.float32)

    q_ref[0] = q
    ind_ref[0, 0] = ind

    part = jnp.sum((q - x) ** 2).reshape(1, 1)

    @pl.when(b == 0)
    def _init():
        dsum_ref[...] = jnp.zeros((1, 1), jnp.float32)

    dsum_ref[...] += part

    @pl.when(b == B - 1)
    def _finish():
        dsum_ref[...] = dsum_ref[...] / TOTAL


def kernel(input, embeddings):
    x = input.reshape(B, DIM, NTOK)
    q, ind, dsum = pl.pallas_call(
        _vq_body,
        grid=(B,),
        in_specs=[
            pl.BlockSpec((1, DIM, NTOK), lambda b: (b, 0, 0)),
            pl.BlockSpec((DIM, N_EMBED), lambda b: (0, 0)),
        ],
        out_specs=[
            pl.BlockSpec((1, DIM, NTOK), lambda b: (b, 0, 0)),
            pl.BlockSpec((1, 1, NTOK), lambda b: (b, 0, 0)),
            pl.BlockSpec((1, 1), lambda b: (0, 0)),
        ],
        out_shape=[
            jax.ShapeDtypeStruct((B, DIM, NTOK), jnp.float32),
            jax.ShapeDtypeStruct((B, 1, NTOK), jnp.int32),
            jax.ShapeDtypeStruct((1, 1), jnp.float32),
        ],
    )(x, embeddings)
    quantize_st = q.reshape(B, DIM, H, W)
    embed_ind = ind.reshape(B, H, W)
    diff = dsum.reshape(())
    return quantize_st, diff, embed_ind


# speedup vs baseline: 2.3277x; 1.1597x over previous
import jax
import jax.numpy as jnp
from jax.experimental import pallas as pl

DIM = 64
N_EMBED = 1024
B, H, W = 16, 32, 32
NTOK = H * W
TOTAL = B * DIM * H * W


def _vq_body(x_ref, e_ref, q_ref, ind_ref, dsum_ref):
    b = pl.program_id(0)
    x = x_ref[0]
    e = e_ref[...]

    et = e.T
    x2 = jnp.sum(x * x, axis=0, keepdims=True)
    e2t = jnp.sum(et * et, axis=1, keepdims=True)
    prodt = jnp.dot(et, x, preferred_element_type=jnp.float32)
    dist = x2 - 2.0 * prodt + e2t
    ind = jnp.argmin(dist, axis=0).astype(jnp.int32)

    code_ids = jax.lax.broadcasted_iota(jnp.int32, (N_EMBED, NTOK), 0)
    onehot = (code_ids == ind[None, :]).astype(jnp.float32)
    q = jax.lax.dot(e, onehot,
                    preferred_element_type=jnp.float32)

    q_ref[0] = q
    ind_ref[0, 0] = ind

    part = jnp.sum((q - x) ** 2).reshape(1, 1)

    @pl.when(b == 0)
    def _init():
        dsum_ref[...] = jnp.zeros((1, 1), jnp.float32)

    dsum_ref[...] += part

    @pl.when(b == B - 1)
    def _finish():
        dsum_ref[...] = dsum_ref[...] / TOTAL


def kernel(input, embeddings):
    x = input.reshape(B, DIM, NTOK)
    q, ind, dsum = pl.pallas_call(
        _vq_body,
        grid=(B,),
        in_specs=[
            pl.BlockSpec((1, DIM, NTOK), lambda b: (b, 0, 0)),
            pl.BlockSpec((DIM, N_EMBED), lambda b: (0, 0)),
        ],
        out_specs=[
            pl.BlockSpec((1, DIM, NTOK), lambda b: (b, 0, 0)),
            pl.BlockSpec((1, 1, NTOK), lambda b: (b, 0, 0)),
            pl.BlockSpec((1, 1), lambda b: (0, 0)),
        ],
        out_shape=[
            jax.ShapeDtypeStruct((B, DIM, NTOK), jnp.float32),
            jax.ShapeDtypeStruct((B, 1, NTOK), jnp.int32),
            jax.ShapeDtypeStruct((1, 1), jnp.float32),
        ],
    )(x, embeddings)
    quantize_st = q.reshape(B, DIM, H, W)
    embed_ind = ind.reshape(B, H, W)
    diff = dsum.reshape(())
    return quantize_st, diff, embed_ind


# speedup vs baseline: 2.5109x; 1.0787x over previous
import jax
import jax.numpy as jnp
from jax.experimental import pallas as pl

DIM = 64
N_EMBED = 1024
B, H, W = 16, 32, 32
NTOK = H * W
TOTAL = B * DIM * H * W


BPS = 2
GRID = B // BPS


def _vq_body(x_ref, e_ref, q_ref, ind_ref, dsum_ref):
    g = pl.program_id(0)
    e = e_ref[...]

    et = e.T
    e2t = jnp.sum(et * et, axis=1, keepdims=True)

    part = jnp.zeros((1, 1), jnp.float32)
    for i in range(BPS):
        x = x_ref[i]
        x2 = jnp.sum(x * x, axis=0, keepdims=True)
        prodt = jnp.dot(et, x, preferred_element_type=jnp.float32)
        dist = x2 - 2.0 * prodt + e2t
        ind = jnp.argmin(dist, axis=0).astype(jnp.int32)

        code_ids = jax.lax.broadcasted_iota(jnp.int32, (N_EMBED, NTOK), 0)
        onehot = (code_ids == ind[None, :]).astype(jnp.float32)
        q = jax.lax.dot(e, onehot,
                        preferred_element_type=jnp.float32)

        q_ref[i] = q
        ind_ref[i, 0] = ind
        part = part + jnp.sum((q - x) ** 2).reshape(1, 1)

    @pl.when(g == 0)
    def _init():
        dsum_ref[...] = jnp.zeros((1, 1), jnp.float32)

    dsum_ref[...] += part

    @pl.when(g == GRID - 1)
    def _finish():
        dsum_ref[...] = dsum_ref[...] / TOTAL


def kernel(input, embeddings):
    x = input.reshape(B, DIM, NTOK)
    q, ind, dsum = pl.pallas_call(
        _vq_body,
        grid=(GRID,),
        in_specs=[
            pl.BlockSpec((BPS, DIM, NTOK), lambda g: (g, 0, 0)),
            pl.BlockSpec((DIM, N_EMBED), lambda g: (0, 0)),
        ],
        out_specs=[
            pl.BlockSpec((BPS, DIM, NTOK), lambda g: (g, 0, 0)),
            pl.BlockSpec((BPS, 1, NTOK), lambda g: (g, 0, 0)),
            pl.BlockSpec((1, 1), lambda g: (0, 0)),
        ],
        out_shape=[
            jax.ShapeDtypeStruct((B, DIM, NTOK), jnp.float32),
            jax.ShapeDtypeStruct((B, 1, NTOK), jnp.int32),
            jax.ShapeDtypeStruct((1, 1), jnp.float32),
        ],
    )(x, embeddings)
    quantize_st = q.reshape(B, DIM, H, W)
    embed_ind = ind.reshape(B, H, W)
    diff = dsum.reshape(())
    return quantize_st, diff, embed_ind


# speedup vs baseline: 2.6126x; 1.0405x over previous
import jax
import jax.numpy as jnp
from jax.experimental import pallas as pl

DIM = 64
N_EMBED = 1024
B, H, W = 16, 32, 32
NTOK = H * W
TOTAL = B * DIM * H * W


BPS = 4
GRID = B // BPS


def _vq_body(x_ref, e_ref, q_ref, ind_ref, dsum_ref):
    g = pl.program_id(0)
    e = e_ref[...]

    et = e.T
    e2t = jnp.sum(et * et, axis=1, keepdims=True)

    part = jnp.zeros((1, 1), jnp.float32)
    for i in range(BPS):
        x = x_ref[i]
        x2 = jnp.sum(x * x, axis=0, keepdims=True)
        prodt = jnp.dot(et, x, preferred_element_type=jnp.float32)
        dist = x2 - 2.0 * prodt + e2t
        ind = jnp.argmin(dist, axis=0).astype(jnp.int32)

        code_ids = jax.lax.broadcasted_iota(jnp.int32, (N_EMBED, NTOK), 0)
        onehot = (code_ids == ind[None, :]).astype(jnp.float32)
        q = jax.lax.dot(e, onehot,
                        preferred_element_type=jnp.float32)

        q_ref[i] = q
        ind_ref[i, 0] = ind
        part = part + jnp.sum((q - x) ** 2).reshape(1, 1)

    @pl.when(g == 0)
    def _init():
        dsum_ref[...] = jnp.zeros((1, 1), jnp.float32)

    dsum_ref[...] += part

    @pl.when(g == GRID - 1)
    def _finish():
        dsum_ref[...] = dsum_ref[...] / TOTAL


def kernel(input, embeddings):
    x = input.reshape(B, DIM, NTOK)
    q, ind, dsum = pl.pallas_call(
        _vq_body,
        grid=(GRID,),
        in_specs=[
            pl.BlockSpec((BPS, DIM, NTOK), lambda g: (g, 0, 0)),
            pl.BlockSpec((DIM, N_EMBED), lambda g: (0, 0)),
        ],
        out_specs=[
            pl.BlockSpec((BPS, DIM, NTOK), lambda g: (g, 0, 0)),
            pl.BlockSpec((BPS, 1, NTOK), lambda g: (g, 0, 0)),
            pl.BlockSpec((1, 1), lambda g: (0, 0)),
        ],
        out_shape=[
            jax.ShapeDtypeStruct((B, DIM, NTOK), jnp.float32),
            jax.ShapeDtypeStruct((B, 1, NTOK), jnp.int32),
            jax.ShapeDtypeStruct((1, 1), jnp.float32),
        ],
    )(x, embeddings)
    quantize_st = q.reshape(B, DIM, H, W)
    embed_ind = ind.reshape(B, H, W)
    diff = dsum.reshape(())
    return quantize_st, diff, embed_ind


# speedup vs baseline: 2.6492x; 1.0140x over previous
import jax
import jax.numpy as jnp
from jax.experimental import pallas as pl

DIM = 64
N_EMBED = 1024
B, H, W = 16, 32, 32
NTOK = H * W
TOTAL = B * DIM * H * W


BPS = 8
GRID = B // BPS


def _vq_body(x_ref, e_ref, q_ref, ind_ref, dsum_ref):
    g = pl.program_id(0)
    e = e_ref[...]

    et = e.T
    e2t = jnp.sum(et * et, axis=1, keepdims=True)

    part = jnp.zeros((1, 1), jnp.float32)
    for i in range(BPS):
        x = x_ref[i]
        x2 = jnp.sum(x * x, axis=0, keepdims=True)
        prodt = jnp.dot(et, x, preferred_element_type=jnp.float32)
        dist = x2 - 2.0 * prodt + e2t
        ind = jnp.argmin(dist, axis=0).astype(jnp.int32)

        code_ids = jax.lax.broadcasted_iota(jnp.int32, (N_EMBED, NTOK), 0)
        onehot = (code_ids == ind[None, :]).astype(jnp.float32)
        q = jax.lax.dot(e, onehot,
                        preferred_element_type=jnp.float32)

        q_ref[i] = q
        ind_ref[i, 0] = ind
        part = part + jnp.sum((q - x) ** 2).reshape(1, 1)

    @pl.when(g == 0)
    def _init():
        dsum_ref[...] = jnp.zeros((1, 1), jnp.float32)

    dsum_ref[...] += part

    @pl.when(g == GRID - 1)
    def _finish():
        dsum_ref[...] = dsum_ref[...] / TOTAL


def kernel(input, embeddings):
    x = input.reshape(B, DIM, NTOK)
    q, ind, dsum = pl.pallas_call(
        _vq_body,
        grid=(GRID,),
        in_specs=[
            pl.BlockSpec((BPS, DIM, NTOK), lambda g: (g, 0, 0)),
            pl.BlockSpec((DIM, N_EMBED), lambda g: (0, 0)),
        ],
        out_specs=[
            pl.BlockSpec((BPS, DIM, NTOK), lambda g: (g, 0, 0)),
            pl.BlockSpec((BPS, 1, NTOK), lambda g: (g, 0, 0)),
            pl.BlockSpec((1, 1), lambda g: (0, 0)),
        ],
        out_shape=[
            jax.ShapeDtypeStruct((B, DIM, NTOK), jnp.float32),
            jax.ShapeDtypeStruct((B, 1, NTOK), jnp.int32),
            jax.ShapeDtypeStruct((1, 1), jnp.float32),
        ],
    )(x, embeddings)
    quantize_st = q.reshape(B, DIM, H, W)
    embed_ind = ind.reshape(B, H, W)
    diff = dsum.reshape(())
    return quantize_st, diff, embed_ind
